# baseline (speedup 1.0000x reference)
import functools

import jax
import jax.numpy as jnp
from jax.experimental import pallas as pl
from jax.experimental.pallas import tpu as pltpu
from jax.experimental.pallas import tpu_sc as plsc

N = 4096
KNN = 8
BLK = 512
NB_ROWS = N // BLK
GW = 128
PADW = 128
SUBW = 16


def _select_kernel(x_cols_ref, x_rows_ref, idx_ref, *, base):
    xt = x_cols_ref[0]
    xr = x_rows_ref[0]
    sqc = jnp.sum(xt * xt, axis=0, keepdims=True)
    dot = jax.lax.dot_general(
        xr, xt, (((0,), (0,)), ((), ())),
        preferred_element_type=jnp.float32,
        precision=jax.lax.Precision.HIGHEST)
    score = sqc - 2.0 * dot

    bits = jax.lax.bitcast_convert_type(score, jnp.int32)
    key = bits ^ ((bits >> 31) & jnp.int32(0x7FFFFFFF))
    colid = jax.lax.broadcasted_iota(jnp.int32, (BLK, N), 1)
    key = ((key + jnp.int32(2048)) & jnp.int32(-4096)) | colid

    MAXK = jnp.int32(0x7FFFFFFF)
    MARK = jnp.int32(0x7FFFFFFE)
    rowid = (jax.lax.broadcasted_iota(jnp.int32, (BLK, N), 0)
             + pl.program_id(0) * BLK)
    key = jnp.where(colid == rowid, MAXK, key)

    b0 = key[:, 0:128]
    b1 = jnp.full((BLK, 128), MARK, jnp.int32)
    b2 = b1
    b3 = b1
    for v in range(1, N // 128):
        x = key[:, 128 * v:128 * (v + 1)]
        h = jnp.maximum(b0, x)
        b0 = jnp.minimum(b0, x)
        h, b1 = jnp.maximum(b1, h), jnp.minimum(b1, h)
        h, b2 = jnp.maximum(b2, h), jnp.minimum(b2, h)
        b3 = jnp.minimum(b3, h)

    for r in range(KNN):
        m = jnp.min(b0, axis=1, keepdims=True)
        idx_ref[0, :, r] = (m[:, 0] & jnp.int32(4095)) + base
        oh = b0 == m
        b0 = jnp.where(oh, b1, b0)
        b1 = jnp.where(oh, b2, b1)
        b2 = jnp.where(oh, b3, b2)
        b3 = jnp.where(oh, MARK, b3)

    @pl.when(jnp.any(b0 == MARK))
    def _():
        k2 = key
        for t in range(KNN):
            m2 = jnp.min(k2, axis=1, keepdims=True)
            idx_ref[0, :, t] = (m2[:, 0] & jnp.int32(4095)) + base
            k2 = jnp.where(k2 == m2, MAXK, k2)


def _finish_kernel(g_ref, x_ref, s_ref, out_ref):
    b = pl.program_id(0)
    c = g_ref[0]
    x = x_ref[0]
    diff = x - c * (1.0 / KNN)
    d2 = jnp.sum(diff * diff, axis=1, keepdims=True)
    dtc = jnp.sqrt(d2)
    m = jnp.max(dtc, axis=0, keepdims=True)
    num = jnp.sum(s_ref[0] * dtc, axis=0, keepdims=True)
    per = num / (m + 1e-6)

    @pl.when(b == 0)
    def _():
        out_ref[...] = jnp.zeros_like(out_ref)

    out_ref[...] += per


def _sc_gather_sum(xpad_flat, idx_flat):
    num_idx = idx_flat.shape[1]
    mesh = plsc.VectorSubcoreMesh(core_axis_name="core",
                                  subcore_axis_name="subcore")

    @pl.kernel(out_type=jax.ShapeDtypeStruct((num_idx // KNN, SUBW),
                                             jnp.float32),
               mesh=mesh,
               scratch_types=[pltpu.VMEM((GW, PADW), jnp.float32)])
    def kern(x_hbm, i_hbm, o_hbm, tmp_ref):
        def body(i_vmem, o_vmem):
            pltpu.sync_copy(x_hbm.at[i_vmem.at[0]], tmp_ref)
            for p in range(GW // KNN):
                acc = tmp_ref[pl.ds(KNN * p, 1), pl.ds(0, SUBW)]
                for j in range(1, KNN):
                    acc = acc + tmp_ref[pl.ds(KNN * p + j, 1), pl.ds(0, SUBW)]
                o_vmem[pl.ds(p, 1), pl.ds(0, SUBW)] = acc

        pltpu.emit_pipeline(
            body,
            grid=(num_idx // GW,),
            in_specs=[pl.BlockSpec((1, GW), index_map=lambda i: (0, i))],
            out_specs=[pl.BlockSpec((GW // KNN, SUBW),
                                    index_map=lambda i: (i, 0))],
            core_axis_name=("core", "subcore"),
            dimension_semantics=(pltpu.PARALLEL,),
        )(i_hbm, o_hbm)

    return kern(xpad_flat, idx_flat)


@jax.jit
def kernel(xyz, spatial_score):
    bs = xyz.shape[0]
    x_t = xyz[:, :, :, 0]
    s = spatial_score[:, 0, :, 0]

    xpad = jnp.concatenate(
        [x_t.transpose(0, 2, 1),
         jnp.zeros((bs, N, PADW - 3), jnp.float32)], axis=2)
    xpad_flat = xpad.reshape(bs * N, PADW)

    sums = []
    for b in range(bs):
        idx_b = pl.pallas_call(
            functools.partial(_select_kernel, base=b * N),
            grid=(NB_ROWS,),
            in_specs=[
                pl.BlockSpec((1, 3, N), lambda i: (0, 0, 0)),
                pl.BlockSpec((1, 3, BLK), lambda i: (0, 0, i)),
            ],
            out_specs=pl.BlockSpec((1, BLK, KNN), lambda i: (i, 0, 0)),
            out_shape=jax.ShapeDtypeStruct((NB_ROWS, BLK, KNN), jnp.int32),
        )(x_t[b:b + 1], x_t[b:b + 1])
        sums.append(_sc_gather_sum(xpad_flat, idx_b.reshape(1, N * KNN)))

    gsum = jnp.stack(sums, axis=0)
    loss = pl.pallas_call(
        _finish_kernel,
        grid=(bs,),
        in_specs=[
            pl.BlockSpec((1, N, SUBW), lambda b: (b, 0, 0)),
            pl.BlockSpec((1, N, SUBW), lambda b: (b, 0, 0)),
            pl.BlockSpec((1, N, 1), lambda b: (b, 0, 0)),
        ],
        out_specs=pl.BlockSpec((1, 1), lambda b: (0, 0)),
        out_shape=jax.ShapeDtypeStruct((1, 1), jnp.float32),
    )(gsum, xpad[:, :, 0:SUBW], s.reshape(bs, N, 1))
    return (loss / (bs * N)).reshape(())


# speedup vs baseline: 1.0201x; 1.0201x over previous
import functools

import jax
import jax.numpy as jnp
from jax.experimental import pallas as pl
from jax.experimental.pallas import tpu as pltpu
from jax.experimental.pallas import tpu_sc as plsc

N = 4096
KNN = 8
BLK = 256
NB_ROWS = N // BLK
GW = 128
PADW = 128
SUBW = 16


def _select_kernel(x_cols_ref, x_rows_ref, idx_ref, *, base):
    xt = x_cols_ref[0]
    xr = x_rows_ref[0]
    sqc = jnp.sum(xt * xt, axis=0, keepdims=True)
    dot = jax.lax.dot_general(
        xr, xt, (((0,), (0,)), ((), ())),
        preferred_element_type=jnp.float32,
        precision=jax.lax.Precision.HIGHEST)
    score = sqc - 2.0 * dot

    bits = jax.lax.bitcast_convert_type(score, jnp.int32)
    key = bits ^ ((bits >> 31) & jnp.int32(0x7FFFFFFF))
    colid = jax.lax.broadcasted_iota(jnp.int32, (BLK, N), 1)
    key = ((key + jnp.int32(2048)) & jnp.int32(-4096)) | colid

    MAXK = jnp.int32(0x7FFFFFFF)
    MARK = jnp.int32(0x7FFFFFFE)
    rowid = (jax.lax.broadcasted_iota(jnp.int32, (BLK, N), 0)
             + pl.program_id(0) * BLK)
    key = jnp.where(colid == rowid, MAXK, key)

    b0 = key[:, 0:128]
    b1 = jnp.full((BLK, 128), MARK, jnp.int32)
    b2 = b1
    b3 = b1
    for v in range(1, N // 128):
        x = key[:, 128 * v:128 * (v + 1)]
        h = jnp.maximum(b0, x)
        b0 = jnp.minimum(b0, x)
        h, b1 = jnp.maximum(b1, h), jnp.minimum(b1, h)
        h, b2 = jnp.maximum(b2, h), jnp.minimum(b2, h)
        b3 = jnp.minimum(b3, h)

    for r in range(KNN):
        m = jnp.min(b0, axis=1, keepdims=True)
        idx_ref[0, :, r] = (m[:, 0] & jnp.int32(4095)) + base
        oh = b0 == m
        b0 = jnp.where(oh, b1, b0)
        b1 = jnp.where(oh, b2, b1)
        b2 = jnp.where(oh, b3, b2)
        b3 = jnp.where(oh, MARK, b3)

    @pl.when(jnp.any(b0 == MARK))
    def _():
        k2 = key
        for t in range(KNN):
            m2 = jnp.min(k2, axis=1, keepdims=True)
            idx_ref[0, :, t] = (m2[:, 0] & jnp.int32(4095)) + base
            k2 = jnp.where(k2 == m2, MAXK, k2)


def _finish_kernel(g_ref, x_ref, s_ref, out_ref):
    b = pl.program_id(0)
    c = g_ref[0]
    x = x_ref[0]
    diff = x - c * (1.0 / KNN)
    d2 = jnp.sum(diff * diff, axis=1, keepdims=True)
    dtc = jnp.sqrt(d2)
    m = jnp.max(dtc, axis=0, keepdims=True)
    num = jnp.sum(s_ref[0] * dtc, axis=0, keepdims=True)
    per = num / (m + 1e-6)

    @pl.when(b == 0)
    def _():
        out_ref[...] = jnp.zeros_like(out_ref)

    out_ref[...] += per


def _sc_gather_sum(xpad_flat, idx_flat):
    num_idx = idx_flat.shape[1]
    mesh = plsc.VectorSubcoreMesh(core_axis_name="core",
                                  subcore_axis_name="subcore")

    @pl.kernel(out_type=jax.ShapeDtypeStruct((num_idx // KNN, SUBW),
                                             jnp.float32),
               mesh=mesh,
               scratch_types=[pltpu.VMEM((GW, PADW), jnp.float32)])
    def kern(x_hbm, i_hbm, o_hbm, tmp_ref):
        def body(i_vmem, o_vmem):
            pltpu.sync_copy(x_hbm.at[i_vmem.at[0]], tmp_ref)
            for p in range(GW // KNN):
                acc = tmp_ref[pl.ds(KNN * p, 1), pl.ds(0, SUBW)]
                for j in range(1, KNN):
                    acc = acc + tmp_ref[pl.ds(KNN * p + j, 1), pl.ds(0, SUBW)]
                o_vmem[pl.ds(p, 1), pl.ds(0, SUBW)] = acc

        pltpu.emit_pipeline(
            body,
            grid=(num_idx // GW,),
            in_specs=[pl.BlockSpec((1, GW), index_map=lambda i: (0, i))],
            out_specs=[pl.BlockSpec((GW // KNN, SUBW),
                                    index_map=lambda i: (i, 0))],
            core_axis_name=("core", "subcore"),
            dimension_semantics=(pltpu.PARALLEL,),
        )(i_hbm, o_hbm)

    return kern(xpad_flat, idx_flat)


@jax.jit
def kernel(xyz, spatial_score):
    bs = xyz.shape[0]
    x_t = xyz[:, :, :, 0]
    s = spatial_score[:, 0, :, 0]

    xpad = jnp.concatenate(
        [x_t.transpose(0, 2, 1),
         jnp.zeros((bs, N, PADW - 3), jnp.float32)], axis=2)
    xpad_flat = xpad.reshape(bs * N, PADW)

    sums = []
    for b in range(bs):
        idx_b = pl.pallas_call(
            functools.partial(_select_kernel, base=b * N),
            grid=(NB_ROWS,),
            in_specs=[
                pl.BlockSpec((1, 3, N), lambda i: (0, 0, 0)),
                pl.BlockSpec((1, 3, BLK), lambda i: (0, 0, i)),
            ],
            out_specs=pl.BlockSpec((1, BLK, KNN), lambda i: (i, 0, 0)),
            out_shape=jax.ShapeDtypeStruct((NB_ROWS, BLK, KNN), jnp.int32),
        )(x_t[b:b + 1], x_t[b:b + 1])
        sums.append(_sc_gather_sum(xpad_flat, idx_b.reshape(1, N * KNN)))

    gsum = jnp.stack(sums, axis=0)
    loss = pl.pallas_call(
        _finish_kernel,
        grid=(bs,),
        in_specs=[
            pl.BlockSpec((1, N, SUBW), lambda b: (b, 0, 0)),
            pl.BlockSpec((1, N, SUBW), lambda b: (b, 0, 0)),
            pl.BlockSpec((1, N, 1), lambda b: (b, 0, 0)),
        ],
        out_specs=pl.BlockSpec((1, 1), lambda b: (0, 0)),
        out_shape=jax.ShapeDtypeStruct((1, 1), jnp.float32),
    )(gsum, xpad[:, :, 0:SUBW], s.reshape(bs, N, 1))
    return (loss / (bs * N)).reshape(())
